# baseline (speedup 1.0000x reference)
import functools

import jax
import jax.numpy as jnp
from jax.experimental import pallas as pl
from jax.experimental.pallas import tpu as pltpu


def _x2h_kernel(x_ref, w_ref, b_ref, o_ref, *, Bb, Tt):
    xt = jnp.transpose(x_ref[...], (1, 0, 2)).reshape(Tt * Bb, -1)
    r = jax.lax.dot_general(
        xt.astype(jnp.bfloat16),
        w_ref[...],
        (((1,), (1,)), ((), ())),
        preferred_element_type=jnp.float32,
    )
    r = r + b_ref[...]
    o_ref[...] = r.reshape(Tt, Bb, -1)


def _scan_kernel(xh_ref, wh_ref, wy_ref, by_ref, y_ref, h_ref, *, U, num_steps):
    j = pl.program_id(1)

    @pl.when(j == 0)
    def _():
        h_ref[...] = jnp.zeros_like(h_ref)

    h = h_ref[...]
    for u in range(U):
        hw = jnp.dot(
            h.astype(jnp.bfloat16), wh_ref[...], preferred_element_type=jnp.float32
        )
        h = jnp.maximum(xh_ref[u] + hw, 0.0)
    h_ref[...] = h

    @pl.when(j == num_steps - 1)
    def _():
        y_ref[...] = (
            jnp.dot(
                h.astype(jnp.bfloat16), wy_ref[...], preferred_element_type=jnp.float32
            )
            + by_ref[...]
        )


@jax.jit
def kernel(x, W_x2h, b_x2h, W_h2h, W_h2y, b_h2y):
    B, T, D_in = x.shape
    D_h = W_h2h.shape[0]
    D_out = W_h2y.shape[0]

    Wxb = W_x2h.astype(jnp.bfloat16)
    Whb = W_h2h.T.astype(jnp.bfloat16)
    Wyb = W_h2y.T.astype(jnp.bfloat16)
    bx = b_x2h.reshape(1, D_h)
    by = b_h2y.reshape(1, D_out)

    Bb = min(8, B)
    Tt = min(128, T)
    xh = pl.pallas_call(
        functools.partial(_x2h_kernel, Bb=Bb, Tt=Tt),
        grid=(B // Bb, T // Tt),
        in_specs=[
            pl.BlockSpec((Bb, Tt, D_in), lambda i, j: (i, j, 0)),
            pl.BlockSpec((D_h, D_in), lambda i, j: (0, 0)),
            pl.BlockSpec((1, D_h), lambda i, j: (0, 0)),
        ],
        out_specs=pl.BlockSpec((Tt, Bb, D_h), lambda i, j: (j, i, 0)),
        out_shape=jax.ShapeDtypeStruct((T, B, D_h), jnp.float32),
        compiler_params=pltpu.CompilerParams(
            dimension_semantics=("parallel", "arbitrary"),
            vmem_limit_bytes=56 * 1024 * 1024,
        ),
    )(x, Wxb, bx)

    U = 16 if T % 16 == 0 else 1
    num_steps = T // U
    NC = 2 if B % 16 == 0 else 1
    Bc = B // NC
    y = pl.pallas_call(
        functools.partial(_scan_kernel, U=U, num_steps=num_steps),
        grid=(NC, num_steps),
        in_specs=[
            pl.BlockSpec((U, Bc, D_h), lambda c, j: (j, c, 0)),
            pl.BlockSpec((D_h, D_h), lambda c, j: (0, 0)),
            pl.BlockSpec((D_h, D_out), lambda c, j: (0, 0)),
            pl.BlockSpec((1, D_out), lambda c, j: (0, 0)),
        ],
        out_specs=pl.BlockSpec((Bc, D_out), lambda c, j: (c, 0)),
        out_shape=jax.ShapeDtypeStruct((B, D_out), jnp.float32),
        scratch_shapes=[pltpu.VMEM((Bc, D_h), jnp.float32)],
        compiler_params=pltpu.CompilerParams(
            dimension_semantics=("parallel", "arbitrary"),
            vmem_limit_bytes=56 * 1024 * 1024,
        ),
    )(xh, Whb, Wyb, by)
    return y


# speedup vs baseline: 1.7735x; 1.7735x over previous
import functools

import jax
import jax.numpy as jnp
from jax.experimental import pallas as pl
from jax.experimental.pallas import tpu as pltpu


def _x2h_kernel(x_ref, w_ref, b_ref, o_ref, *, Bb, Tt):
    xt = jnp.transpose(x_ref[...], (1, 0, 2)).reshape(Tt * Bb, -1)
    r = jax.lax.dot_general(
        xt.astype(jnp.bfloat16),
        w_ref[...],
        (((1,), (1,)), ((), ())),
        preferred_element_type=jnp.float32,
    )
    r = r + b_ref[...]
    o_ref[...] = r.reshape(Tt, Bb, -1).astype(jnp.bfloat16)


def _scan_kernel(xh_ref, wh_ref, wy_ref, by_ref, y_ref, h_ref, *, U, num_steps):
    j = pl.program_id(0)

    @pl.when(j == 0)
    def _():
        h_ref[...] = jnp.zeros_like(h_ref)

    h = h_ref[...]
    for u in range(U):
        hw = jnp.dot(
            h.astype(jnp.bfloat16), wh_ref[...], preferred_element_type=jnp.float32
        )
        h = jnp.maximum(xh_ref[u].astype(jnp.float32) + hw, 0.0)
    h_ref[...] = h

    @pl.when(j == num_steps - 1)
    def _():
        y_ref[...] = (
            jnp.dot(
                h.astype(jnp.bfloat16), wy_ref[...], preferred_element_type=jnp.float32
            )
            + by_ref[...]
        )


@jax.jit
def kernel(x, W_x2h, b_x2h, W_h2h, W_h2y, b_h2y):
    B, T, D_in = x.shape
    D_h = W_h2h.shape[0]
    D_out = W_h2y.shape[0]

    Wxb = W_x2h.astype(jnp.bfloat16)
    Whb = W_h2h.T.astype(jnp.bfloat16)
    Wyb = W_h2y.T.astype(jnp.bfloat16)
    bx = b_x2h.reshape(1, D_h)
    by = b_h2y.reshape(1, D_out)

    Bb = min(8, B)
    Tt = min(128, T)
    xh = pl.pallas_call(
        functools.partial(_x2h_kernel, Bb=Bb, Tt=Tt),
        grid=(B // Bb, T // Tt),
        in_specs=[
            pl.BlockSpec((Bb, Tt, D_in), lambda i, j: (i, j, 0)),
            pl.BlockSpec((D_h, D_in), lambda i, j: (0, 0)),
            pl.BlockSpec((1, D_h), lambda i, j: (0, 0)),
        ],
        out_specs=pl.BlockSpec((Tt, Bb, D_h), lambda i, j: (j, i, 0)),
        out_shape=jax.ShapeDtypeStruct((T, B, D_h), jnp.bfloat16),
        compiler_params=pltpu.CompilerParams(
            dimension_semantics=("parallel", "arbitrary"),
            vmem_limit_bytes=56 * 1024 * 1024,
        ),
    )(x, Wxb, bx)

    U = 16 if T % 16 == 0 else 1
    num_steps = T // U
    y = pl.pallas_call(
        functools.partial(_scan_kernel, U=U, num_steps=num_steps),
        grid=(num_steps,),
        in_specs=[
            pl.BlockSpec((U, B, D_h), lambda j: (j, 0, 0)),
            pl.BlockSpec((D_h, D_h), lambda j: (0, 0)),
            pl.BlockSpec((D_h, D_out), lambda j: (0, 0)),
            pl.BlockSpec((1, D_out), lambda j: (0, 0)),
        ],
        out_specs=pl.BlockSpec((B, D_out), lambda j: (0, 0)),
        out_shape=jax.ShapeDtypeStruct((B, D_out), jnp.float32),
        scratch_shapes=[pltpu.VMEM((B, D_h), jnp.float32)],
        compiler_params=pltpu.CompilerParams(
            dimension_semantics=("arbitrary",),
            vmem_limit_bytes=56 * 1024 * 1024,
        ),
    )(xh, Whb, Wyb, by)
    return y
